# baseline (speedup 1.0000x reference)
import jax
import jax.numpy as jnp
from jax import lax
from jax.experimental import pallas as pl
from jax.experimental.pallas import tpu as pltpu

_PROJ = 128
_BASE = 10000.0
_SBLK = 256


def _rope_kernel(sid_ref, x_ref, o_ref):
    lane = lax.broadcasted_iota(jnp.int32, (_SBLK, _PROJ), 1)
    pair = (lane // 2).astype(jnp.float32)
    theta = jnp.exp(pair * (-2.0 * jnp.log(_BASE) / _PROJ))
    sid = sid_ref[0, 0, 0, :]
    m = sid[:, None] * theta
    c = jnp.cos(m)
    s = jnp.sin(m)
    row = lax.broadcasted_iota(jnp.int32, (_PROJ, _PROJ), 0)
    col = lax.broadcasted_iota(jnp.int32, (_PROJ, _PROJ), 1)
    pval = jnp.where(col % 2 == 0, -1.0, 1.0)
    perm = jnp.where(row == (col ^ 1), pval, 0.0).astype(jnp.bfloat16)
    x = x_ref[0]
    x_rot = lax.dot_general(
        x.astype(jnp.bfloat16), perm,
        (((2,), (0,)), ((), ())),
        preferred_element_type=jnp.float32,
    )
    o_ref[0] = c[None] * x + s[None] * x_rot


def kernel(x, seq_id):
    b, h1, h2, seq, proj = x.shape
    heads = h1 * h2
    xf = x.reshape(b, heads, seq, proj)
    nblk = seq // _SBLK
    sid = seq_id.reshape(b, nblk, 1, _SBLK).astype(jnp.float32)
    out = pl.pallas_call(
        _rope_kernel,
        grid=(b, nblk),
        in_specs=[
            pl.BlockSpec((1, 1, 1, _SBLK), lambda i, j: (i, j, 0, 0)),
            pl.BlockSpec((1, heads, _SBLK, proj), lambda i, j: (i, 0, j, 0)),
        ],
        out_specs=pl.BlockSpec((1, heads, _SBLK, proj), lambda i, j: (i, 0, j, 0)),
        out_shape=jax.ShapeDtypeStruct((b, heads, seq, proj), x.dtype),
    )(sid, xf)
    return out.reshape(x.shape)
